# Initial kernel scaffold; baseline (speedup 1.0000x reference)
import jax
import jax.numpy as jnp
from jax.experimental import pallas as pl


def kernel(x, local_indices, group_mask, pe):
    raise NotImplementedError("write your pallas kernel here")


# speedup vs baseline: 1.3700x; 1.3700x over previous
import jax
import jax.numpy as jnp
from jax.experimental import pallas as pl

GROUP = 64
ROWS = 512


def _body(idx_ref, msk_ref, x_ref, pe_ref, o_ref):
    idx = idx_ref[0]
    msk = msk_ref[0]
    onehot = (idx == jax.lax.broadcasted_iota(jnp.int32, (ROWS, GROUP), 1))
    gathered = jax.lax.dot_general(
        onehot.astype(jnp.float32), pe_ref[...],
        (((1,), (0,)), ((), ())),
        preferred_element_type=jnp.float32,
        precision=jax.lax.Precision.HIGHEST,
    )
    o_ref[...] = jnp.where(msk > 0, gathered, x_ref[...])


def kernel(x, local_indices, group_mask, pe):
    b, s, d = x.shape
    n = b * s
    nb = n // ROWS
    x2 = x.reshape(n, d)
    idx3 = local_indices.reshape(nb, ROWS, 1)
    msk3 = group_mask.astype(jnp.int32).reshape(nb, ROWS, 1)
    out = pl.pallas_call(
        _body,
        grid=(nb,),
        in_specs=[
            pl.BlockSpec((1, ROWS, 1), lambda i: (i, 0, 0)),
            pl.BlockSpec((1, ROWS, 1), lambda i: (i, 0, 0)),
            pl.BlockSpec((ROWS, d), lambda i: (i, 0)),
            pl.BlockSpec((GROUP, d), lambda i: (0, 0)),
        ],
        out_specs=pl.BlockSpec((ROWS, d), lambda i: (i, 0)),
        out_shape=jax.ShapeDtypeStruct((n, d), x.dtype),
    )(idx3, msk3, x2, pe)
    return out.reshape(b, s, d)


# speedup vs baseline: 1.5276x; 1.1150x over previous
import jax
import jax.numpy as jnp
from jax.experimental import pallas as pl

GROUP = 64
ROWS = 1024


def _body(idx_ref, msk_ref, x_ref, pe_ref, o_ref):
    idx = idx_ref[0]
    msk = msk_ref[0]
    onehot = (idx == jax.lax.broadcasted_iota(jnp.int32, (ROWS, GROUP), 1))
    gathered = jax.lax.dot_general(
        onehot.astype(jnp.float32), pe_ref[...],
        (((1,), (0,)), ((), ())),
        preferred_element_type=jnp.float32,
        precision=jax.lax.Precision.HIGHEST,
    )
    o_ref[...] = jnp.where(msk > 0, gathered, x_ref[...])


def kernel(x, local_indices, group_mask, pe):
    b, s, d = x.shape
    n = b * s
    nb = n // ROWS
    x2 = x.reshape(n, d)
    idx3 = local_indices.reshape(nb, ROWS, 1)
    msk3 = group_mask.astype(jnp.int32).reshape(nb, ROWS, 1)
    out = pl.pallas_call(
        _body,
        grid=(nb,),
        in_specs=[
            pl.BlockSpec((1, ROWS, 1), lambda i: (i, 0, 0)),
            pl.BlockSpec((1, ROWS, 1), lambda i: (i, 0, 0)),
            pl.BlockSpec((ROWS, d), lambda i: (i, 0)),
            pl.BlockSpec((GROUP, d), lambda i: (0, 0)),
        ],
        out_specs=pl.BlockSpec((ROWS, d), lambda i: (i, 0)),
        out_shape=jax.ShapeDtypeStruct((n, d), x.dtype),
    )(idx3, msk3, x2, pe)
    return out.reshape(b, s, d)


# speedup vs baseline: 1.5884x; 1.0398x over previous
import jax
import jax.numpy as jnp
from jax.experimental import pallas as pl

GROUP = 64
ROWS = 2048


def _body(idx_ref, msk_ref, x_ref, pe_ref, o_ref):
    idx = idx_ref[0]
    msk = msk_ref[0]
    onehot = (idx == jax.lax.broadcasted_iota(jnp.int32, (ROWS, GROUP), 1))
    gathered = jax.lax.dot_general(
        onehot.astype(jnp.float32), pe_ref[...],
        (((1,), (0,)), ((), ())),
        preferred_element_type=jnp.float32,
        precision=jax.lax.Precision.HIGHEST,
    )
    o_ref[...] = jnp.where(msk > 0, gathered, x_ref[...])


def kernel(x, local_indices, group_mask, pe):
    b, s, d = x.shape
    n = b * s
    nb = n // ROWS
    x2 = x.reshape(n, d)
    idx3 = local_indices.reshape(nb, ROWS, 1)
    msk3 = group_mask.astype(jnp.int32).reshape(nb, ROWS, 1)
    out = pl.pallas_call(
        _body,
        grid=(nb,),
        in_specs=[
            pl.BlockSpec((1, ROWS, 1), lambda i: (i, 0, 0)),
            pl.BlockSpec((1, ROWS, 1), lambda i: (i, 0, 0)),
            pl.BlockSpec((ROWS, d), lambda i: (i, 0)),
            pl.BlockSpec((GROUP, d), lambda i: (0, 0)),
        ],
        out_specs=pl.BlockSpec((ROWS, d), lambda i: (i, 0)),
        out_shape=jax.ShapeDtypeStruct((n, d), x.dtype),
    )(idx3, msk3, x2, pe)
    return out.reshape(b, s, d)


# speedup vs baseline: 1.8731x; 1.1792x over previous
import jax
import jax.numpy as jnp
from jax.experimental import pallas as pl

GROUP = 64
ROWS = 2048


def _body(idx_ref, msk_ref, x_ref, pe_ref, o_ref):
    idx = idx_ref[0]
    msk = msk_ref[0]
    onehot = (idx == jax.lax.broadcasted_iota(jnp.int32, (ROWS, GROUP), 1))
    gathered = jax.lax.dot_general(
        onehot.astype(jnp.bfloat16), pe_ref[...].astype(jnp.bfloat16),
        (((1,), (0,)), ((), ())),
        preferred_element_type=jnp.float32,
    )
    o_ref[...] = jnp.where(msk > 0, gathered, x_ref[...])


def kernel(x, local_indices, group_mask, pe):
    b, s, d = x.shape
    n = b * s
    nb = n // ROWS
    x2 = x.reshape(n, d)
    idx3 = local_indices.reshape(nb, ROWS, 1)
    msk3 = group_mask.astype(jnp.int32).reshape(nb, ROWS, 1)
    out = pl.pallas_call(
        _body,
        grid=(nb,),
        in_specs=[
            pl.BlockSpec((1, ROWS, 1), lambda i: (i, 0, 0)),
            pl.BlockSpec((1, ROWS, 1), lambda i: (i, 0, 0)),
            pl.BlockSpec((ROWS, d), lambda i: (i, 0)),
            pl.BlockSpec((GROUP, d), lambda i: (0, 0)),
        ],
        out_specs=pl.BlockSpec((ROWS, d), lambda i: (i, 0)),
        out_shape=jax.ShapeDtypeStruct((n, d), x.dtype),
    )(idx3, msk3, x2, pe)
    return out.reshape(b, s, d)
